# baseline (speedup 1.0000x reference)
import functools

import jax
import jax.numpy as jnp
from jax import lax
from jax.experimental import pallas as pl
from jax.experimental.pallas import tpu as pltpu
from jax.experimental.pallas import tpu_sc as plsc

V = 1000
VP = 1024
N = 51200
NC, NS = 2, 16
NW = NC * NS
BPW = N // NW
CHUNK = 80
NCHUNK = BPW // CHUNK
L = 16


def _tc_lse_table(embedding):
    def body(x_ref, o_ref):
        rows = x_ref[...]
        m = jnp.max(rows, axis=1, keepdims=True)
        s = jnp.sum(jnp.exp(rows - m), axis=1, keepdims=True)
        o_ref[...] = m + jnp.log(s)

    return pl.pallas_call(
        body,
        in_specs=[pl.BlockSpec((V, V), lambda: (0, 0))],
        out_specs=pl.BlockSpec((V, 1), lambda: (0, 0)),
        out_shape=jax.ShapeDtypeStruct((V, 1), jnp.float32),
    )(embedding)


def _sc_gather_and_nll(table_p, lse_flat, idx_flat, tgt_flat):
    mesh = plsc.VectorSubcoreMesh(core_axis_name="c", subcore_axis_name="s")

    @functools.partial(
        pl.kernel,
        out_type=[
            jax.ShapeDtypeStruct((N, VP), jnp.float32),
            jax.ShapeDtypeStruct((NW, L), jnp.float32),
        ],
        mesh=mesh,
        compiler_params=pltpu.CompilerParams(needs_layout_passes=False),
        scratch_types=[
            pltpu.VMEM((BPW,), jnp.int32),
            pltpu.VMEM((BPW,), jnp.int32),
            pltpu.VMEM((V,), jnp.float32),
            pltpu.VMEM((CHUNK, VP), jnp.float32),
            pltpu.VMEM((L,), jnp.float32),
            pltpu.SemaphoreType.DMA,
            pltpu.SemaphoreType.DMA,
        ],
    )
    def k(table_hbm, lse_hbm, idx_hbm, tgt_hbm, out_hbm, part_hbm,
          idx_v, tgt_v, lse_v, buf, acc_v, sem_g, sem_w):
        wid = lax.axis_index("s") * NC + lax.axis_index("c")
        base = wid * BPW
        pltpu.sync_copy(idx_hbm.at[pl.ds(base, BPW)], idx_v)
        pltpu.sync_copy(tgt_hbm.at[pl.ds(base, BPW)], tgt_v)
        pltpu.sync_copy(lse_hbm, lse_v)
        acc_v[...] = jnp.zeros((L,), jnp.float32)
        row_iota = lax.iota(jnp.int32, L)

        @pl.loop(0, NCHUNK)
        def _(c):
            off = c * CHUNK

            @pl.when(c > 0)
            def _():
                pltpu.make_async_copy(
                    buf, out_hbm.at[pl.ds(base + off - CHUNK, CHUNK)], sem_w
                ).wait()

            pltpu.async_copy(
                table_hbm.at[idx_v.at[pl.ds(off, CHUNK)]], buf, sem_g
            ).wait()
            pltpu.async_copy(
                buf, out_hbm.at[pl.ds(base + off, CHUNK)], sem_w)

            @pl.loop(0, CHUNK, step=L)
            def _(j):
                t_reg = tgt_v[pl.ds(off + j, L)]
                i_reg = idx_v[pl.ds(off + j, L)]
                vals = plsc.load_gather(buf, [row_iota + j, t_reg])
                lses = plsc.load_gather(lse_v, [i_reg])
                acc_v[...] = acc_v[...] + (lses - vals)

        pltpu.make_async_copy(
            buf, out_hbm.at[pl.ds(base + BPW - CHUNK, CHUNK)], sem_w
        ).wait()
        pltpu.sync_copy(acc_v, part_hbm.at[wid])

    return k(table_p, lse_flat, idx_flat, tgt_flat)


BLK = 512
G = N // BLK


def _tc_narrow(padded):
    def body(x_ref, o_ref):
        o_ref[...] = x_ref[...][:, :V]

    return pl.pallas_call(
        body,
        grid=(G,),
        compiler_params=pltpu.CompilerParams(
            dimension_semantics=("parallel",)),
        in_specs=[pl.BlockSpec((BLK, VP), lambda i: (i, 0))],
        out_specs=pl.BlockSpec((BLK, V), lambda i: (i, 0)),
        out_shape=jax.ShapeDtypeStruct((N, V), jnp.float32),
    )(padded)


def kernel(idx, targets, embedding):
    idx_flat = idx.reshape(-1)
    tgt_flat = targets.reshape(-1)
    emb_p = jnp.pad(embedding, ((0, 0), (0, VP - V)),
                    constant_values=jnp.float32(-1e30))
    lse_flat = _tc_lse_table(embedding).reshape(-1)
    out_p, parts = _sc_gather_and_nll(emb_p, lse_flat, idx_flat, tgt_flat)
    return _tc_narrow(out_p), jnp.sum(parts) / jnp.float32(N)


# speedup vs baseline: 1.5488x; 1.5488x over previous
import functools

import jax
import jax.numpy as jnp
from jax import lax
from jax.experimental import pallas as pl
from jax.experimental.pallas import tpu as pltpu
from jax.experimental.pallas import tpu_sc as plsc

V = 1000
VP = 1024
N = 51200
NC, NS = 2, 16
NW = NC * NS
BPW = N // NW
CHUNK = 80
NCHUNK = BPW // CHUNK
L = 16


def _tc_prep(embedding):
    def body(x_ref, op_ref, ol_ref):
        rows = x_ref[...]
        op_ref[:, :V] = rows
        m = jnp.max(rows, axis=1, keepdims=True)
        s = jnp.sum(jnp.exp(rows - m), axis=1, keepdims=True)
        ol_ref[...] = m + jnp.log(s)

    return pl.pallas_call(
        body,
        in_specs=[pl.BlockSpec((V, V), lambda: (0, 0))],
        out_specs=[
            pl.BlockSpec((V, VP), lambda: (0, 0)),
            pl.BlockSpec((V, 1), lambda: (0, 0)),
        ],
        out_shape=[
            jax.ShapeDtypeStruct((V, VP), jnp.float32),
            jax.ShapeDtypeStruct((V, 1), jnp.float32),
        ],
    )(embedding)


def _sc_gather_and_nll(table_p, lse_flat, idx_flat, tgt_flat):
    mesh = plsc.VectorSubcoreMesh(core_axis_name="c", subcore_axis_name="s")

    @functools.partial(
        pl.kernel,
        out_type=[
            jax.ShapeDtypeStruct((N, VP), jnp.float32),
            jax.ShapeDtypeStruct((NW, L), jnp.float32),
        ],
        mesh=mesh,
        compiler_params=pltpu.CompilerParams(needs_layout_passes=False),
        scratch_types=[
            pltpu.VMEM((BPW,), jnp.int32),
            pltpu.VMEM((BPW,), jnp.int32),
            pltpu.VMEM((V,), jnp.float32),
            pltpu.VMEM((CHUNK, VP), jnp.float32),
            pltpu.VMEM((L,), jnp.float32),
            pltpu.SemaphoreType.DMA,
            pltpu.SemaphoreType.DMA,
        ],
    )
    def k(table_hbm, lse_hbm, idx_hbm, tgt_hbm, out_hbm, part_hbm,
          idx_v, tgt_v, lse_v, buf, acc_v, sem_g, sem_w):
        wid = lax.axis_index("s") * NC + lax.axis_index("c")
        base = wid * BPW
        pltpu.sync_copy(idx_hbm.at[pl.ds(base, BPW)], idx_v)
        pltpu.sync_copy(tgt_hbm.at[pl.ds(base, BPW)], tgt_v)
        pltpu.sync_copy(lse_hbm, lse_v)
        acc_v[...] = jnp.zeros((L,), jnp.float32)
        row_iota = lax.iota(jnp.int32, L)

        @pl.loop(0, NCHUNK)
        def _(c):
            off = c * CHUNK

            @pl.when(c > 0)
            def _():
                pltpu.make_async_copy(
                    buf, out_hbm.at[pl.ds(base + off - CHUNK, CHUNK)], sem_w
                ).wait()

            pltpu.async_copy(
                table_hbm.at[idx_v.at[pl.ds(off, CHUNK)]], buf, sem_g
            ).wait()
            pltpu.async_copy(
                buf, out_hbm.at[pl.ds(base + off, CHUNK)], sem_w)

            @pl.loop(0, CHUNK, step=L)
            def _(j):
                t_reg = tgt_v[pl.ds(off + j, L)]
                i_reg = idx_v[pl.ds(off + j, L)]
                vals = plsc.load_gather(buf, [row_iota + j, t_reg])
                lses = plsc.load_gather(lse_v, [i_reg])
                acc_v[...] = acc_v[...] + (lses - vals)

        pltpu.make_async_copy(
            buf, out_hbm.at[pl.ds(base + BPW - CHUNK, CHUNK)], sem_w
        ).wait()
        pltpu.sync_copy(acc_v, part_hbm.at[wid])

    return k(table_p, lse_flat, idx_flat, tgt_flat)


def kernel(idx, targets, embedding):
    idx_flat = idx.reshape(-1)
    tgt_flat = targets.reshape(-1)
    emb_p, lse_col = _tc_prep(embedding)
    lse_flat = lse_col.reshape(-1)
    out_p, parts = _sc_gather_and_nll(emb_p, lse_flat, idx_flat, tgt_flat)
    return out_p[:, :V], jnp.sum(parts) / jnp.float32(N)
